# baseline (speedup 1.0000x reference)
import jax
import jax.numpy as jnp
from jax.experimental import pallas as pl
from jax.experimental.pallas import tpu as pltpu

_EPS = 1e-5
_VMEM_LIMIT = 48 * 1024 * 1024


def _row_stats_kernel(x_ref, st_ref):
    x = x_ref[...]
    s = jnp.sum(x, axis=(0, 2, 3))
    ss = jnp.sum(x * x, axis=(0, 2, 3))
    st_ref[...] = jnp.stack([s, ss], axis=1)[None]


def _silu_apply_kernel(x_ref, ss_ref, o_ref):
    x = x_ref[...]
    scale = ss_ref[0, :, 0][None, :, None, None]
    shift = ss_ref[0, :, 1][None, :, None, None]
    z = x * scale + shift
    e = jnp.exp(-jnp.maximum(z, -80.0))
    d = 1.0 + e
    r = pl.reciprocal(d, approx=True)
    r = r * (2.0 - d * r)
    o_ref[...] = z * r


def kernel(x_nchw, gamma, beta):
    N, C, H, W = x_nchw.shape
    cnt = N * H * W
    grid = (N,)

    stats = pl.pallas_call(
        _row_stats_kernel,
        out_shape=jax.ShapeDtypeStruct((N, C, 2), jnp.float32),
        grid=grid,
        in_specs=[pl.BlockSpec((1, C, H, W), lambda n: (n, 0, 0, 0))],
        out_specs=pl.BlockSpec((1, C, 2), lambda n: (n, 0, 0)),
        compiler_params=pltpu.CompilerParams(
            dimension_semantics=("parallel",),
            vmem_limit_bytes=_VMEM_LIMIT),
    )(x_nchw)

    sum_c = jnp.sum(stats[:, :, 0], axis=0)
    ssq_c = jnp.sum(stats[:, :, 1], axis=0)
    mean_c = sum_c / cnt
    var_c = ssq_c / cnt - mean_c * mean_c
    inv_std = jax.lax.rsqrt(var_c + _EPS)
    scale_c = gamma.astype(jnp.float32) * inv_std
    shift_c = beta.astype(jnp.float32) - mean_c * scale_c
    ss_nc = jnp.broadcast_to(
        jnp.stack([scale_c, shift_c], axis=1)[None], (N, C, 2))

    out = pl.pallas_call(
        _silu_apply_kernel,
        out_shape=jax.ShapeDtypeStruct((N, C, H, W), jnp.float32),
        grid=grid,
        in_specs=[pl.BlockSpec((1, C, H, W), lambda n: (n, 0, 0, 0)),
                  pl.BlockSpec((1, C, 2), lambda n: (n, 0, 0))],
        out_specs=pl.BlockSpec((1, C, H, W), lambda n: (n, 0, 0, 0)),
        compiler_params=pltpu.CompilerParams(
            dimension_semantics=("parallel",),
            vmem_limit_bytes=_VMEM_LIMIT),
    )(x_nchw, ss_nc)

    return out
